# baseline (speedup 1.0000x reference)
import functools

import jax
import jax.numpy as jnp
from jax import lax
from jax.experimental import pallas as pl
from jax.experimental.pallas import tpu as pltpu
from jax.experimental.pallas import tpu_sc as plsc

_T_OUT = 8192
_T_IN = 4096
_BR = 256

_NS = 16
_PER_TILE = _T_OUT // _NS
_SLICE = _T_IN // _NS


def _argmax_body(x_ref, out_ref):
    x = x_ref[...]
    rowmax = jnp.max(x, axis=1, keepdims=True)
    col = jax.lax.broadcasted_iota(jnp.int32, x.shape, 1)
    out_ref[...] = jnp.min(jnp.where(x == rowmax, col, _T_IN), axis=1,
                           keepdims=True)


def _tc_argmax(att_ws):
    return pl.pallas_call(
        _argmax_body,
        grid=(_T_OUT // _BR,),
        in_specs=[pl.BlockSpec((_BR, _T_IN), lambda i: (i, 0))],
        out_specs=pl.BlockSpec((_BR, 1), lambda i: (i, 0)),
        out_shape=jax.ShapeDtypeStruct((_T_OUT, 1), jnp.int32),
    )(att_ws)


def _sc_hist_body(idx_hbm, out_hbm, idx_v, ones_v, zeros_v, hist_sh):
    sid = lax.axis_index("s")
    one = jnp.ones((16,), jnp.int32)
    zero = jnp.zeros((16,), jnp.int32)
    for k in range(128 // 16):
        ones_v[pl.ds(16 * k, 16)] = one
    for k in range(_SLICE // 16):
        zeros_v[pl.ds(16 * k, 16)] = zero
    pltpu.sync_copy(zeros_v, hist_sh.at[pl.ds(sid * _SLICE, _SLICE)])
    plsc.subcore_barrier()
    for j in range(_PER_TILE // 128):
        pltpu.sync_copy(idx_hbm.at[sid * (_PER_TILE // 128) + j], idx_v)
        pltpu.sync_copy(ones_v, hist_sh.at[idx_v], add=True)
    plsc.subcore_barrier()
    pltpu.sync_copy(hist_sh.at[pl.ds(sid * _SLICE, _SLICE)],
                    out_hbm.at[pl.ds(sid * _SLICE, _SLICE)])


_sc_hist = functools.partial(
    pl.kernel,
    out_type=jax.ShapeDtypeStruct((_T_IN,), jnp.int32),
    mesh=plsc.VectorSubcoreMesh(core_axis_name="c", subcore_axis_name="s",
                                num_cores=1),
    scratch_types=[
        pltpu.VMEM((128,), jnp.int32),
        pltpu.VMEM((128,), jnp.int32),
        pltpu.VMEM((_SLICE,), jnp.int32),
        pltpu.VMEM_SHARED((_T_IN,), jnp.int32),
    ],
)(_sc_hist_body)


def kernel(att_ws):
    idx = _tc_argmax(att_ws).reshape(_T_OUT // 128, 128)
    return _sc_hist(idx)


# speedup vs baseline: 1.0704x; 1.0704x over previous
import functools

import jax
import jax.numpy as jnp
from jax import lax
from jax.experimental import pallas as pl
from jax.experimental.pallas import tpu as pltpu
from jax.experimental.pallas import tpu_sc as plsc

_T_OUT = 8192
_T_IN = 4096
_BR = 256

_NS = 16
_PER_TILE = _T_OUT // _NS
_SLICE = _T_IN // _NS


def _argmax_body(x_ref, out_ref):
    x = x_ref[...]
    rowmax = jnp.max(x, axis=1, keepdims=True)
    col = jax.lax.broadcasted_iota(jnp.int32, x.shape, 1)
    first = jnp.min(jnp.where(x == rowmax, col, _T_IN), axis=1)
    out_ref[...] = first.reshape(1, _BR // 128, 128)


def _tc_argmax(att_ws):
    return pl.pallas_call(
        _argmax_body,
        grid=(_T_OUT // _BR,),
        in_specs=[pl.BlockSpec((_BR, _T_IN), lambda i: (i, 0))],
        out_specs=pl.BlockSpec((1, _BR // 128, 128), lambda i: (i, 0, 0)),
        out_shape=jax.ShapeDtypeStruct(
            (_T_OUT // _BR, _BR // 128, 128), jnp.int32),
    )(att_ws)


def _sc_hist_body(idx_hbm, out_hbm, idx_v, ones_v, zeros_v, hist_sh):
    sid = lax.axis_index("s")
    one = jnp.ones((16,), jnp.int32)
    zero = jnp.zeros((16,), jnp.int32)
    for k in range(128 // 16):
        ones_v[pl.ds(16 * k, 16)] = one
    for k in range(_SLICE // 16):
        zeros_v[pl.ds(16 * k, 16)] = zero
    pltpu.sync_copy(zeros_v, hist_sh.at[pl.ds(sid * _SLICE, _SLICE)])
    plsc.subcore_barrier()
    for j in range(_PER_TILE // 128):
        pltpu.sync_copy(idx_hbm.at[sid * (_PER_TILE // 128) + j], idx_v)
        pltpu.sync_copy(ones_v, hist_sh.at[idx_v], add=True)
    plsc.subcore_barrier()
    pltpu.sync_copy(hist_sh.at[pl.ds(sid * _SLICE, _SLICE)],
                    out_hbm.at[pl.ds(sid * _SLICE, _SLICE)])


_sc_hist = functools.partial(
    pl.kernel,
    out_type=jax.ShapeDtypeStruct((_T_IN,), jnp.int32),
    mesh=plsc.VectorSubcoreMesh(core_axis_name="c", subcore_axis_name="s",
                                num_cores=1),
    scratch_types=[
        pltpu.VMEM((128,), jnp.int32),
        pltpu.VMEM((128,), jnp.int32),
        pltpu.VMEM((_SLICE,), jnp.int32),
        pltpu.VMEM_SHARED((_T_IN,), jnp.int32),
    ],
)(_sc_hist_body)


def kernel(att_ws):
    idx = _tc_argmax(att_ws).reshape(_T_OUT // 128, 128)
    return _sc_hist(idx)


# speedup vs baseline: 1.2465x; 1.1645x over previous
import functools

import jax
import jax.numpy as jnp
from jax import lax
from jax.experimental import pallas as pl
from jax.experimental.pallas import tpu as pltpu
from jax.experimental.pallas import tpu_sc as plsc

_T_OUT = 8192
_T_IN = 4096
_BR = 512

_NS = 16
_PER_TILE = _T_OUT // _NS
_CHUNKS = _PER_TILE // 128
_SLICE = _T_IN // _NS


def _argmax_body(x_ref, out_ref):
    x = x_ref[...]
    rowmax = jnp.max(x, axis=1, keepdims=True)
    col = jax.lax.broadcasted_iota(jnp.int32, x.shape, 1)
    first = jnp.min(jnp.where(x == rowmax, col, _T_IN), axis=1)
    out_ref[...] = first.reshape(1, _BR // 128, 128)


def _tc_argmax(att_ws):
    return pl.pallas_call(
        _argmax_body,
        grid=(_T_OUT // _BR,),
        in_specs=[pl.BlockSpec((_BR, _T_IN), lambda i: (i, 0))],
        out_specs=pl.BlockSpec((1, _BR // 128, 128), lambda i: (i, 0, 0)),
        out_shape=jax.ShapeDtypeStruct(
            (_T_OUT // _BR, _BR // 128, 128), jnp.int32),
    )(att_ws)


def _sc_hist_body(idx_hbm, out_hbm, i0, i1, i2, i3, ones_v, zeros_v,
                  hist_sh, sem):
    sid = lax.axis_index("s")
    idx_vs = [i0, i1, i2, i3]
    copies = []
    for j in range(_CHUNKS):
        copies.append(
            pltpu.async_copy(idx_hbm.at[sid * _CHUNKS + j], idx_vs[j], sem))
    one = jnp.ones((16,), jnp.int32)
    zero = jnp.zeros((16,), jnp.int32)
    for k in range(128 // 16):
        ones_v[pl.ds(16 * k, 16)] = one
    for k in range(_SLICE // 16):
        zeros_v[pl.ds(16 * k, 16)] = zero
    pltpu.sync_copy(zeros_v, hist_sh.at[pl.ds(sid * _SLICE, _SLICE)])
    for c in copies:
        c.wait()
    plsc.subcore_barrier()
    adds = []
    for j in range(_CHUNKS):
        adds.append(
            pltpu.async_copy(ones_v, hist_sh.at[idx_vs[j]], sem, add=True))
    for a in adds:
        a.wait()
    plsc.subcore_barrier()
    pltpu.sync_copy(hist_sh.at[pl.ds(sid * _SLICE, _SLICE)],
                    out_hbm.at[pl.ds(sid * _SLICE, _SLICE)])


_sc_hist = functools.partial(
    pl.kernel,
    out_type=jax.ShapeDtypeStruct((_T_IN,), jnp.int32),
    mesh=plsc.VectorSubcoreMesh(core_axis_name="c", subcore_axis_name="s",
                                num_cores=1),
    scratch_types=[
        pltpu.VMEM((128,), jnp.int32),
        pltpu.VMEM((128,), jnp.int32),
        pltpu.VMEM((128,), jnp.int32),
        pltpu.VMEM((128,), jnp.int32),
        pltpu.VMEM((128,), jnp.int32),
        pltpu.VMEM((_SLICE,), jnp.int32),
        pltpu.VMEM_SHARED((_T_IN,), jnp.int32),
        pltpu.SemaphoreType.DMA,
    ],
)(_sc_hist_body)


def kernel(att_ws):
    idx = _tc_argmax(att_ws).reshape(_T_OUT // 128, 128)
    return _sc_hist(idx)


# speedup vs baseline: 1.2571x; 1.0085x over previous
import functools

import jax
import jax.numpy as jnp
from jax import lax
from jax.experimental import pallas as pl
from jax.experimental.pallas import tpu as pltpu
from jax.experimental.pallas import tpu_sc as plsc

_T_OUT = 8192
_T_IN = 4096
_BR = 1024

_NS = 16
_PER_TILE = _T_OUT // _NS
_CHUNKS = _PER_TILE // 128
_SLICE = _T_IN // _NS


def _argmax_body(x_ref, out_ref):
    x = x_ref[...]
    rowmax = jnp.max(x, axis=1, keepdims=True)
    col = jax.lax.broadcasted_iota(jnp.int32, x.shape, 1)
    first = jnp.min(jnp.where(x == rowmax, col, _T_IN), axis=1)
    out_ref[...] = first.reshape(1, _BR // 128, 128)


def _tc_argmax(att_ws):
    return pl.pallas_call(
        _argmax_body,
        grid=(_T_OUT // _BR,),
        in_specs=[pl.BlockSpec((_BR, _T_IN), lambda i: (i, 0))],
        out_specs=pl.BlockSpec((1, _BR // 128, 128), lambda i: (i, 0, 0)),
        out_shape=jax.ShapeDtypeStruct(
            (_T_OUT // _BR, _BR // 128, 128), jnp.int32),
    )(att_ws)


def _sc_hist_body(idx_hbm, out_hbm, i0, i1, i2, i3, ones_v, zeros_v,
                  hist_sh, sem):
    sid = lax.axis_index("s")
    idx_vs = [i0, i1, i2, i3]
    copies = []
    for j in range(_CHUNKS):
        copies.append(
            pltpu.async_copy(idx_hbm.at[sid * _CHUNKS + j], idx_vs[j], sem))
    one = jnp.ones((16,), jnp.int32)
    zero = jnp.zeros((16,), jnp.int32)
    for k in range(128 // 16):
        ones_v[pl.ds(16 * k, 16)] = one
    for k in range(_SLICE // 16):
        zeros_v[pl.ds(16 * k, 16)] = zero
    pltpu.sync_copy(zeros_v, hist_sh.at[pl.ds(sid * _SLICE, _SLICE)])
    for c in copies:
        c.wait()
    plsc.subcore_barrier()
    adds = []
    for j in range(_CHUNKS):
        adds.append(
            pltpu.async_copy(ones_v, hist_sh.at[idx_vs[j]], sem, add=True))
    for a in adds:
        a.wait()
    plsc.subcore_barrier()
    pltpu.sync_copy(hist_sh.at[pl.ds(sid * _SLICE, _SLICE)],
                    out_hbm.at[pl.ds(sid * _SLICE, _SLICE)])


_sc_hist = functools.partial(
    pl.kernel,
    out_type=jax.ShapeDtypeStruct((_T_IN,), jnp.int32),
    mesh=plsc.VectorSubcoreMesh(core_axis_name="c", subcore_axis_name="s",
                                num_cores=1),
    scratch_types=[
        pltpu.VMEM((128,), jnp.int32),
        pltpu.VMEM((128,), jnp.int32),
        pltpu.VMEM((128,), jnp.int32),
        pltpu.VMEM((128,), jnp.int32),
        pltpu.VMEM((128,), jnp.int32),
        pltpu.VMEM((_SLICE,), jnp.int32),
        pltpu.VMEM_SHARED((_T_IN,), jnp.int32),
        pltpu.SemaphoreType.DMA,
    ],
)(_sc_hist_body)


def kernel(att_ws):
    idx = _tc_argmax(att_ws).reshape(_T_OUT // 128, 128)
    return _sc_hist(idx)


# speedup vs baseline: 1.2773x; 1.0161x over previous
import functools

import jax
import jax.numpy as jnp
from jax import lax
from jax.experimental import pallas as pl
from jax.experimental.pallas import tpu as pltpu
from jax.experimental.pallas import tpu_sc as plsc

_T_OUT = 8192
_T_IN = 4096
_BR = 1024

_NS = 16
_PER_TILE = _T_OUT // _NS
_CHUNKS = _PER_TILE // 128
_SLICE = _T_IN // _NS


def _argmax_body(x_ref, out_ref):
    c_w = 128
    mval = x_ref[:, :c_w]
    midx = jnp.zeros((_BR, c_w), jnp.int32)
    for c in range(1, _T_IN // c_w):
        chunk = x_ref[:, pl.ds(c * c_w, c_w)]
        gt = chunk > mval
        mval = jnp.where(gt, chunk, mval)
        midx = jnp.where(gt, c, midx)
    rowmax = jnp.max(mval, axis=1, keepdims=True)
    lane = jax.lax.broadcasted_iota(jnp.int32, (_BR, c_w), 1)
    colcand = midx * c_w + lane
    first = jnp.min(jnp.where(mval == rowmax, colcand, _T_IN), axis=1)
    out_ref[...] = first.reshape(1, _BR // 128, 128)


def _tc_argmax(att_ws):
    return pl.pallas_call(
        _argmax_body,
        grid=(_T_OUT // _BR,),
        in_specs=[pl.BlockSpec((_BR, _T_IN), lambda i: (i, 0))],
        out_specs=pl.BlockSpec((1, _BR // 128, 128), lambda i: (i, 0, 0)),
        out_shape=jax.ShapeDtypeStruct(
            (_T_OUT // _BR, _BR // 128, 128), jnp.int32),
    )(att_ws)


def _sc_hist_body(idx_hbm, out_hbm, i0, i1, i2, i3, ones_v, zeros_v,
                  hist_sh, sem):
    sid = lax.axis_index("s")
    idx_vs = [i0, i1, i2, i3]
    copies = []
    for j in range(_CHUNKS):
        copies.append(
            pltpu.async_copy(idx_hbm.at[sid * _CHUNKS + j], idx_vs[j], sem))
    one = jnp.ones((16,), jnp.int32)
    zero = jnp.zeros((16,), jnp.int32)
    for k in range(128 // 16):
        ones_v[pl.ds(16 * k, 16)] = one
    for k in range(_SLICE // 16):
        zeros_v[pl.ds(16 * k, 16)] = zero
    pltpu.sync_copy(zeros_v, hist_sh.at[pl.ds(sid * _SLICE, _SLICE)])
    for c in copies:
        c.wait()
    plsc.subcore_barrier()
    adds = []
    for j in range(_CHUNKS):
        adds.append(
            pltpu.async_copy(ones_v, hist_sh.at[idx_vs[j]], sem, add=True))
    for a in adds:
        a.wait()
    plsc.subcore_barrier()
    pltpu.sync_copy(hist_sh.at[pl.ds(sid * _SLICE, _SLICE)],
                    out_hbm.at[pl.ds(sid * _SLICE, _SLICE)])


_sc_hist = functools.partial(
    pl.kernel,
    out_type=jax.ShapeDtypeStruct((_T_IN,), jnp.int32),
    mesh=plsc.VectorSubcoreMesh(core_axis_name="c", subcore_axis_name="s",
                                num_cores=1, num_subcores=_NS),
    scratch_types=[
        pltpu.VMEM((128,), jnp.int32),
        pltpu.VMEM((128,), jnp.int32),
        pltpu.VMEM((128,), jnp.int32),
        pltpu.VMEM((128,), jnp.int32),
        pltpu.VMEM((128,), jnp.int32),
        pltpu.VMEM((_SLICE,), jnp.int32),
        pltpu.VMEM_SHARED((_T_IN,), jnp.int32),
        pltpu.SemaphoreType.DMA,
    ],
)(_sc_hist_body)


def kernel(att_ws):
    idx = _tc_argmax(att_ws).reshape(_T_OUT // 128, 128)
    return _sc_hist(idx)


# speedup vs baseline: 1.3168x; 1.0309x over previous
import functools

import jax
import jax.numpy as jnp
from jax import lax
from jax.experimental import pallas as pl
from jax.experimental.pallas import tpu as pltpu
from jax.experimental.pallas import tpu_sc as plsc

_T_OUT = 8192
_T_IN = 4096
_BR = 512

_NS = 16
_PER_TILE = _T_OUT // _NS
_CHUNKS = _PER_TILE // 128
_SLICE = _T_IN // _NS


def _argmax_body(x_ref, out_ref):
    c_w = 128
    mval = x_ref[:, :c_w]
    midx = jnp.zeros((_BR, c_w), jnp.int32)
    for c in range(1, _T_IN // c_w):
        chunk = x_ref[:, pl.ds(c * c_w, c_w)]
        gt = chunk > mval
        mval = jnp.where(gt, chunk, mval)
        midx = jnp.where(gt, c, midx)
    rowmax = jnp.max(mval, axis=1, keepdims=True)
    lane = jax.lax.broadcasted_iota(jnp.int32, (_BR, c_w), 1)
    colcand = midx * c_w + lane
    first = jnp.min(jnp.where(mval == rowmax, colcand, _T_IN), axis=1)
    out_ref[...] = first.reshape(1, _BR // 128, 128)


def _tc_argmax(att_ws):
    return pl.pallas_call(
        _argmax_body,
        grid=(_T_OUT // _BR,),
        in_specs=[pl.BlockSpec((_BR, _T_IN), lambda i: (i, 0))],
        out_specs=pl.BlockSpec((1, _BR // 128, 128), lambda i: (i, 0, 0)),
        out_shape=jax.ShapeDtypeStruct(
            (_T_OUT // _BR, _BR // 128, 128), jnp.int32),
    )(att_ws)


def _sc_hist_body(idx_hbm, out_hbm, i0, i1, i2, i3, ones_v, zeros_v,
                  hist_sh, sem):
    sid = lax.axis_index("s")
    idx_vs = [i0, i1, i2, i3]
    copies = []
    for j in range(_CHUNKS):
        copies.append(
            pltpu.async_copy(idx_hbm.at[sid * _CHUNKS + j], idx_vs[j], sem))
    one = jnp.ones((16,), jnp.int32)
    zero = jnp.zeros((16,), jnp.int32)
    for k in range(128 // 16):
        ones_v[pl.ds(16 * k, 16)] = one
    for k in range(_SLICE // 16):
        zeros_v[pl.ds(16 * k, 16)] = zero
    pltpu.sync_copy(zeros_v, hist_sh.at[pl.ds(sid * _SLICE, _SLICE)])
    for c in copies:
        c.wait()
    plsc.subcore_barrier()
    adds = []
    for j in range(_CHUNKS):
        adds.append(
            pltpu.async_copy(ones_v, hist_sh.at[idx_vs[j]], sem, add=True))
    for a in adds:
        a.wait()
    plsc.subcore_barrier()
    pltpu.sync_copy(hist_sh.at[pl.ds(sid * _SLICE, _SLICE)],
                    out_hbm.at[pl.ds(sid * _SLICE, _SLICE)])


_sc_hist = functools.partial(
    pl.kernel,
    out_type=jax.ShapeDtypeStruct((_T_IN,), jnp.int32),
    mesh=plsc.VectorSubcoreMesh(core_axis_name="c", subcore_axis_name="s",
                                num_cores=1, num_subcores=_NS),
    scratch_types=[
        pltpu.VMEM((128,), jnp.int32),
        pltpu.VMEM((128,), jnp.int32),
        pltpu.VMEM((128,), jnp.int32),
        pltpu.VMEM((128,), jnp.int32),
        pltpu.VMEM((128,), jnp.int32),
        pltpu.VMEM((_SLICE,), jnp.int32),
        pltpu.VMEM_SHARED((_T_IN,), jnp.int32),
        pltpu.SemaphoreType.DMA,
    ],
)(_sc_hist_body)


def kernel(att_ws):
    idx = _tc_argmax(att_ws).reshape(_T_OUT // 128, 128)
    return _sc_hist(idx)
